# Initial kernel scaffold; baseline (speedup 1.0000x reference)
import jax
import jax.numpy as jnp
from jax.experimental import pallas as pl


def kernel(x, adj_t, bases0, comp0, root0, bias0, bases1, comp1, root1, bias1, bases2, comp2, root2, bias2):
    raise NotImplementedError("write your pallas kernel here")


# speedup vs baseline: 2.6700x; 2.6700x over previous
import functools

import jax
import jax.numpy as jnp
from jax import lax
from jax.experimental import pallas as pl
from jax.experimental.pallas import tpu as pltpu
from jax.experimental.pallas import tpu_sc as plsc

N_NODES = 10000
N_EDGES = 320000
D = 128
NUM_RELS = 8
NUM_BASES = 4

BN = 256
NP = 10240
NSEG = NUM_RELS * NP
NBLK = 20
BLK = NSEG // NBLK
NCORES = 2
NSUB = 16
CHUNK = 128
EPT = 20096
NCH = EPT // CHUNK
EPAD = EPT * NSUB
BPC = NBLK // NCORES
TROWS = BLK // NSUB


def _sc_body(x_hbm, src_hbm, seg_hbm, s_out, c_out,
             seg_vm, src_vm, gat_vm, sct_vm, rows_vm, ones_vm, zrow_vm,
             zcnt_vm, blk_sh, cnt_sh):
  c = lax.axis_index("c")
  s = lax.axis_index("s")
  lane = lax.iota(jnp.int32, 16)

  e0 = s * EPT
  pltpu.sync_copy(seg_hbm.at[pl.ds(e0, EPT)], seg_vm)
  pltpu.sync_copy(src_hbm.at[pl.ds(e0, EPT)], src_vm)

  for g in range(CHUNK // 16):
    ones_vm[pl.ds(g * 16, 16)] = jnp.ones((16,), jnp.float32)
  for r in range(64):
    for g in range(D // 16):
      zrow_vm[r, pl.ds(g * 16, 16)] = jnp.zeros((16,), jnp.float32)
  for g in range(512 // 16):
    zcnt_vm[pl.ds(g * 16, 16)] = jnp.zeros((16,), jnp.float32)

  for bi in range(BPC):
    b = 2 * bi + c
    blk_lo = b * BLK
    blk_hi = blk_lo + BLK

    for k in range(TROWS // 64):
      pltpu.sync_copy(zrow_vm, blk_sh.at[pl.ds(s * TROWS + k * 64, 64)])
    pltpu.sync_copy(zcnt_vm.at[pl.ds(0, TROWS)], cnt_sh.at[pl.ds(s * TROWS, TROWS)])

    @pl.when(s == 0)
    def _():
      pltpu.sync_copy(zrow_vm.at[pl.ds(0, 16)], blk_sh.at[pl.ds(BLK, 16)])
      pltpu.sync_copy(zcnt_vm, cnt_sh.at[pl.ds(BLK, 512)])

    plsc.subcore_barrier()

    def chunk_body(ch, carry):
      base = ch * CHUNK
      first = seg_vm[pl.ds(base, 16)][0]
      last = seg_vm[pl.ds(base + CHUNK - 16, 16)][15]

      @pl.when(jnp.logical_and(first < blk_hi, last >= blk_lo))
      def _():
        for g in range(CHUNK // 16):
          sg = seg_vm[pl.ds(base + g * 16, 16)]
          inb = jnp.logical_and(sg >= blk_lo, sg < blk_hi)
          sct_vm[pl.ds(g * 16, 16)] = jnp.where(inb, sg - blk_lo,
                                                BLK + (lane & 7))
          gat_vm[pl.ds(g * 16, 16)] = src_vm[pl.ds(base + g * 16, 16)]
        pltpu.sync_copy(x_hbm.at[gat_vm], rows_vm)
        pltpu.sync_copy(rows_vm, blk_sh.at[sct_vm], add=True)
        pltpu.sync_copy(ones_vm, cnt_sh.at[sct_vm], add=True)

      return carry

    lax.fori_loop(0, NCH, chunk_body, 0)

    plsc.subcore_barrier()
    out0 = b * BLK + s * TROWS
    pltpu.sync_copy(blk_sh.at[pl.ds(s * TROWS, TROWS)],
                    s_out.at[pl.ds(out0, TROWS)])
    pltpu.sync_copy(cnt_sh.at[pl.ds(s * TROWS, TROWS)],
                    c_out.at[pl.ds(out0, TROWS)])
    plsc.subcore_barrier()


_sc_segsum = pl.kernel(
    _sc_body,
    out_type=(jax.ShapeDtypeStruct((NSEG, D), jnp.float32),
              jax.ShapeDtypeStruct((NSEG,), jnp.float32)),
    mesh=plsc.VectorSubcoreMesh(core_axis_name="c", subcore_axis_name="s",
                                num_cores=NCORES, num_subcores=NSUB),
    scratch_types=[
        pltpu.VMEM((EPT,), jnp.int32),
        pltpu.VMEM((EPT,), jnp.int32),
        pltpu.VMEM((CHUNK,), jnp.int32),
        pltpu.VMEM((CHUNK,), jnp.int32),
        pltpu.VMEM((CHUNK, D), jnp.float32),
        pltpu.VMEM((CHUNK,), jnp.float32),
        pltpu.VMEM((64, D), jnp.float32),
        pltpu.VMEM((512,), jnp.float32),
        pltpu.VMEM_SHARED((BLK + 16, D), jnp.float32),
        pltpu.VMEM_SHARED((BLK + 512,), jnp.float32),
    ],
)


def _bf(a):
  return a.astype(jnp.bfloat16).astype(jnp.float32)


def _dense_body(act, x_ref, s_ref, c_ref, bases_ref, comp_ref, root_ref,
                bias_ref, o_ref):
  acc = jnp.dot(_bf(x_ref[...]), _bf(root_ref[...]),
                preferred_element_type=jnp.float32,
                precision=lax.Precision.HIGHEST)
  basesq = [_bf(bases_ref[bb]) for bb in range(NUM_BASES)]
  for r in range(NUM_RELS):
    cr = c_ref[r]
    mr = s_ref[r] * (1.0 / jnp.maximum(cr, 1.0))
    wr = jnp.zeros((D, D), jnp.float32)
    for bb in range(NUM_BASES):
      wr = wr + _bf(comp_ref[r, bb]) * basesq[bb]
    acc = acc + jnp.dot(mr, _bf(wr), preferred_element_type=jnp.float32,
                        precision=lax.Precision.HIGHEST)
  acc = acc + bias_ref[...]
  if act == 0:
    o_ref[...] = _bf(jnp.maximum(acc, 0.0))
  else:
    m = jnp.max(acc, axis=1, keepdims=True)
    e = jnp.exp(acc - m)
    o_ref[...] = e / jnp.sum(e, axis=1, keepdims=True)


def _dense(x, s3, c3, bases, comp, root, bias, act):
  return pl.pallas_call(
      functools.partial(_dense_body, act),
      grid=(NP // BN,),
      in_specs=[
          pl.BlockSpec((BN, D), lambda i: (i, 0)),
          pl.BlockSpec((NUM_RELS, BN, D), lambda i: (0, i, 0)),
          pl.BlockSpec((NUM_RELS, BN, 1), lambda i: (0, i, 0)),
          pl.BlockSpec((NUM_BASES, D, D), lambda i: (0, 0, 0)),
          pl.BlockSpec(memory_space=pltpu.SMEM),
          pl.BlockSpec((D, D), lambda i: (0, 0)),
          pl.BlockSpec((1, D), lambda i: (0, 0)),
      ],
      out_specs=pl.BlockSpec((BN, D), lambda i: (i, 0)),
      out_shape=jax.ShapeDtypeStruct((NP, D), jnp.float32),
  )(x, s3, c3, bases, comp, root, bias)


def kernel(x, adj_t, bases0, comp0, root0, bias0, bases1, comp1, root1,
           bias1, bases2, comp2, root2, bias2):
  src = adj_t[0].astype(jnp.int32)
  dst = adj_t[1].astype(jnp.int32)
  rel = (adj_t[2] % NUM_RELS).astype(jnp.int32)
  seg = rel * NP + dst

  order = jnp.argsort(seg)
  seg_s = jnp.concatenate(
      [seg[order], jnp.full((EPAD - N_EDGES,), NSEG + 7, jnp.int32)])
  src_s = jnp.concatenate(
      [src[order], jnp.arange(EPAD - N_EDGES, dtype=jnp.int32) % 512])

  h = (jnp.zeros((NP, D), jnp.float32).at[:N_NODES].set(x)
       .astype(jnp.bfloat16).astype(jnp.float32))
  layers = [(bases0, comp0, root0, bias0, 0),
            (bases1, comp1, root1, bias1, 0),
            (bases2, comp2, root2, bias2, 1)]
  for bases, comp, root, bias, act in layers:
    s_sum, cnt = _sc_segsum(h, src_s, seg_s)
    h = _dense(h, s_sum.reshape(NUM_RELS, NP, D),
               cnt.reshape(NUM_RELS, NP, 1),
               bases, comp, root, bias.reshape(1, D), act)
  return h[:N_NODES]


# speedup vs baseline: 3.3437x; 1.2523x over previous
import functools

import jax
import jax.numpy as jnp
from jax import lax
from jax.experimental import pallas as pl
from jax.experimental.pallas import tpu as pltpu
from jax.experimental.pallas import tpu_sc as plsc

N_NODES = 10000
N_EDGES = 320000
D = 128
NUM_RELS = 8
NUM_BASES = 4

BN = 256
NP = 10240
NSEG = NUM_RELS * NP
NBLK = 20
BLK = NSEG // NBLK
NCORES = 2
NSUB = 16
CHUNK = 128
SUP = 2
EPT = 20480
NCH = EPT // CHUNK
EPAD = EPT * NSUB
BPC = NBLK // NCORES
TROWS = BLK // NSUB


def _sc_body(x_hbm, src_hbm, seg_hbm, s_out, c_out,
             seg_vm, src_vm, gat_vm, sct_vm, rows_vm, ones_vm, zrow_vm,
             zcnt_vm, blk_sh, cnt_sh, gsems, ssem, csem):
  c = lax.axis_index("c")
  s = lax.axis_index("s")
  lane = lax.iota(jnp.int32, 16)

  e0 = s * EPT
  pltpu.sync_copy(seg_hbm.at[pl.ds(e0, EPT)], seg_vm)
  pltpu.sync_copy(src_hbm.at[pl.ds(e0, EPT)], src_vm)

  for g in range(CHUNK // 16):
    ones_vm[pl.ds(g * 16, 16)] = jnp.ones((16,), jnp.float32)
  for r in range(64):
    for g in range(D // 16):
      zrow_vm[r, pl.ds(g * 16, 16)] = jnp.zeros((16,), jnp.float32)
  for g in range(512 // 16):
    zcnt_vm[pl.ds(g * 16, 16)] = jnp.zeros((16,), jnp.float32)

  for bi in range(BPC):
    b = 2 * bi + c
    blk_lo = b * BLK
    blk_hi = blk_lo + BLK

    for k in range(TROWS // 64):
      pltpu.sync_copy(zrow_vm, blk_sh.at[pl.ds(s * TROWS + k * 64, 64)])
    pltpu.sync_copy(zcnt_vm.at[pl.ds(0, TROWS)], cnt_sh.at[pl.ds(s * TROWS, TROWS)])

    @pl.when(s == 0)
    def _():
      pltpu.sync_copy(zrow_vm.at[pl.ds(0, 16)], blk_sh.at[pl.ds(BLK, 16)])
      pltpu.sync_copy(zcnt_vm, cnt_sh.at[pl.ds(BLK, 512)])

    plsc.subcore_barrier()

    def fired_at(base):
      first = seg_vm[pl.ds(base, 16)][0]
      last = seg_vm[pl.ds(base + CHUNK - 16, 16)][15]
      return jnp.logical_and(first < blk_hi, last >= blk_lo)

    def chunk_group(grp, carry):
      base0 = grp * (CHUNK * SUP)
      for k in range(SUP):
        base = base0 + k * CHUNK

        @pl.when(fired_at(base))
        def _(k=k, base=base):
          for g in range(CHUNK // 16):
            sg = seg_vm[pl.ds(base + g * 16, 16)]
            inb = jnp.logical_and(sg >= blk_lo, sg < blk_hi)
            sct_vm[k, pl.ds(g * 16, 16)] = jnp.where(inb, sg - blk_lo,
                                                     BLK + (lane & 7))
            gat_vm[k, pl.ds(g * 16, 16)] = src_vm[pl.ds(base + g * 16, 16)]
          pltpu.async_copy(x_hbm.at[gat_vm.at[k]], rows_vm.at[k], gsems[k])
          pltpu.async_copy(ones_vm, cnt_sh.at[sct_vm.at[k]], csem, add=True)

      for k in range(SUP):
        base = base0 + k * CHUNK

        @pl.when(fired_at(base))
        def _(k=k):
          pltpu.make_async_copy(x_hbm.at[gat_vm.at[k]], rows_vm.at[k],
                                gsems[k]).wait()
          pltpu.async_copy(rows_vm.at[k], blk_sh.at[sct_vm.at[k]], ssem,
                           add=True)

      for k in range(SUP):
        base = base0 + k * CHUNK

        @pl.when(fired_at(base))
        def _(k=k):
          pltpu.make_async_copy(rows_vm.at[k], blk_sh.at[sct_vm.at[k]],
                                ssem).wait()
          pltpu.make_async_copy(ones_vm, cnt_sh.at[sct_vm.at[k]],
                                csem).wait()

      return carry

    lax.fori_loop(0, NCH // SUP, chunk_group, 0)

    plsc.subcore_barrier()
    out0 = b * BLK + s * TROWS
    pltpu.sync_copy(blk_sh.at[pl.ds(s * TROWS, TROWS)],
                    s_out.at[pl.ds(out0, TROWS)])
    pltpu.sync_copy(cnt_sh.at[pl.ds(s * TROWS, TROWS)],
                    c_out.at[pl.ds(out0, TROWS)])
    plsc.subcore_barrier()


_sc_segsum = pl.kernel(
    _sc_body,
    out_type=(jax.ShapeDtypeStruct((NSEG, D), jnp.float32),
              jax.ShapeDtypeStruct((NSEG,), jnp.float32)),
    mesh=plsc.VectorSubcoreMesh(core_axis_name="c", subcore_axis_name="s",
                                num_cores=NCORES, num_subcores=NSUB),
    scratch_types=[
        pltpu.VMEM((EPT,), jnp.int32),
        pltpu.VMEM((EPT,), jnp.int32),
        pltpu.VMEM((SUP, CHUNK), jnp.int32),
        pltpu.VMEM((SUP, CHUNK), jnp.int32),
        pltpu.VMEM((SUP, CHUNK, D), jnp.float32),
        pltpu.VMEM((CHUNK,), jnp.float32),
        pltpu.VMEM((64, D), jnp.float32),
        pltpu.VMEM((512,), jnp.float32),
        pltpu.VMEM_SHARED((BLK + 16, D), jnp.float32),
        pltpu.VMEM_SHARED((BLK + 512,), jnp.float32),
        [pltpu.SemaphoreType.DMA] * SUP,
        pltpu.SemaphoreType.DMA,
        pltpu.SemaphoreType.DMA,
    ],
)


def _bf(a):
  return a.astype(jnp.bfloat16).astype(jnp.float32)


def _dense_body(act, x_ref, s_ref, c_ref, bases_ref, comp_ref, root_ref,
                bias_ref, o_ref):
  acc = jnp.dot(_bf(x_ref[...]), _bf(root_ref[...]),
                preferred_element_type=jnp.float32,
                precision=lax.Precision.HIGHEST)
  basesq = [_bf(bases_ref[bb]) for bb in range(NUM_BASES)]
  for r in range(NUM_RELS):
    cr = c_ref[r]
    mr = s_ref[r] * (1.0 / jnp.maximum(cr, 1.0))
    wr = jnp.zeros((D, D), jnp.float32)
    for bb in range(NUM_BASES):
      wr = wr + _bf(comp_ref[r, bb]) * basesq[bb]
    acc = acc + jnp.dot(mr, _bf(wr), preferred_element_type=jnp.float32,
                        precision=lax.Precision.HIGHEST)
  acc = acc + bias_ref[...]
  if act == 0:
    o_ref[...] = _bf(jnp.maximum(acc, 0.0))
  else:
    m = jnp.max(acc, axis=1, keepdims=True)
    e = jnp.exp(acc - m)
    o_ref[...] = e / jnp.sum(e, axis=1, keepdims=True)


def _dense(x, s3, c3, bases, comp, root, bias, act):
  return pl.pallas_call(
      functools.partial(_dense_body, act),
      grid=(NP // BN,),
      in_specs=[
          pl.BlockSpec((BN, D), lambda i: (i, 0)),
          pl.BlockSpec((NUM_RELS, BN, D), lambda i: (0, i, 0)),
          pl.BlockSpec((NUM_RELS, BN, 1), lambda i: (0, i, 0)),
          pl.BlockSpec((NUM_BASES, D, D), lambda i: (0, 0, 0)),
          pl.BlockSpec(memory_space=pltpu.SMEM),
          pl.BlockSpec((D, D), lambda i: (0, 0)),
          pl.BlockSpec((1, D), lambda i: (0, 0)),
      ],
      out_specs=pl.BlockSpec((BN, D), lambda i: (i, 0)),
      out_shape=jax.ShapeDtypeStruct((NP, D), jnp.float32),
  )(x, s3, c3, bases, comp, root, bias)


def kernel(x, adj_t, bases0, comp0, root0, bias0, bases1, comp1, root1,
           bias1, bases2, comp2, root2, bias2):
  src = adj_t[0].astype(jnp.int32)
  dst = adj_t[1].astype(jnp.int32)
  rel = (adj_t[2] % NUM_RELS).astype(jnp.int32)
  seg = rel * NP + dst

  order = jnp.argsort(seg)
  seg_s = jnp.concatenate(
      [seg[order], jnp.full((EPAD - N_EDGES,), NSEG + 7, jnp.int32)])
  src_s = jnp.concatenate(
      [src[order], jnp.arange(EPAD - N_EDGES, dtype=jnp.int32) % 512])

  h = (jnp.zeros((NP, D), jnp.float32).at[:N_NODES].set(x)
       .astype(jnp.bfloat16).astype(jnp.float32))
  layers = [(bases0, comp0, root0, bias0, 0),
            (bases1, comp1, root1, bias1, 0),
            (bases2, comp2, root2, bias2, 1)]
  for bases, comp, root, bias, act in layers:
    s_sum, cnt = _sc_segsum(h, src_s, seg_s)
    h = _dense(h, s_sum.reshape(NUM_RELS, NP, D),
               cnt.reshape(NUM_RELS, NP, 1),
               bases, comp, root, bias.reshape(1, D), act)
  return h[:N_NODES]
